# baseline (speedup 1.0000x reference)
import functools

import jax
import jax.numpy as jnp
from jax import lax
from jax.experimental import pallas as pl
from jax.experimental.pallas import tpu as pltpu
from jax.experimental.pallas import tpu_sc as plsc

B = 8
H = 224
W = 224
C = 96
NC = 2
NS = 16
NW = NC * NS
NPIX = B * H * W
PIX_PER_W = NPIX // NW
P = 112
NCHUNK = PIX_PER_W // P
GROUPS = P // 16
CGROUPS = C // 16


def _worker_body(table_hbm, theta_hbm, out_hbm,
                 theta_v, idx00_v, idx01_v, idx10_v, idx11_v,
                 wtop_v, wbot_v, fx_v,
                 r00_v, r01_v, r10_v, r11_v, out_v, sem):
    wid = lax.axis_index("s") * NC + lax.axis_index("c")
    base = wid * PIX_PER_W
    b = base // (H * W)
    bb = b * (H * W)
    local0 = base - bb

    pltpu.sync_copy(theta_hbm, theta_v)
    tb = b * 6

    def bcast_theta(k):
        t = plsc.load_gather(theta_v, [jnp.full((16,), tb + k, jnp.int32)])
        u = plsc.bitcast(t, jnp.int32)
        r = (u + 0x7FFF + ((u >> 16) & 1)) & jnp.int32(-65536)
        return plsc.bitcast(r, jnp.float32)

    t0, t1, t2 = bcast_theta(0), bcast_theta(1), bcast_theta(2)
    t3, t4, t5 = bcast_theta(3), bcast_theta(4), bcast_theta(5)

    lanes_f = lax.iota(jnp.int32, 16).astype(jnp.float32)

    def chunk(k, carry):
        s0 = local0 + k * P
        for g in range(GROUPS):
            s = s0 + g * 16
            i = s // W
            jb = s - i * W
            i_f = jnp.full((16,), i.astype(jnp.float32))
            j_f = jnp.full((16,), jb.astype(jnp.float32)) + lanes_f
            ys = i_f * t0 + j_f * t1 + t2
            xs = i_f * t3 + j_f * t4 + t5
            inb = ((ys >= 0.0) & (ys <= float(H - 1))
                   & (xs >= 0.0) & (xs <= float(W - 1)))
            m = jnp.where(inb, 1.0, 0.0).astype(jnp.float32)
            yc = jnp.minimum(jnp.maximum(ys, 0.0), float(H - 1))
            xc = jnp.minimum(jnp.maximum(xs, 0.0), float(W - 1))
            yb = jnp.minimum(yc.astype(jnp.int32), H - 2)
            xb = jnp.minimum(xc.astype(jnp.int32), W - 2)
            fy = yc - yb.astype(jnp.float32)
            fx = xc - xb.astype(jnp.float32)
            i00 = bb + yb * W + xb
            sl = pl.ds(g * 16, 16)
            idx00_v[sl] = i00
            idx01_v[sl] = i00 + 1
            idx10_v[sl] = i00 + W
            idx11_v[sl] = i00 + W + 1
            wtop_v[sl] = m * (1.0 - fy)
            wbot_v[sl] = m * fy
            fx_v[sl] = fx

        c0 = pltpu.async_copy(table_hbm.at[idx00_v], r00_v, sem)
        c1 = pltpu.async_copy(table_hbm.at[idx01_v], r01_v, sem)
        c2 = pltpu.async_copy(table_hbm.at[idx10_v], r10_v, sem)
        c3 = pltpu.async_copy(table_hbm.at[idx11_v], r11_v, sem)
        c0.wait(); c1.wait(); c2.wait(); c3.wait()

        def combine(p, c):
            pv = jnp.full((16,), p, jnp.int32)
            wt = plsc.load_gather(wtop_v, [pv])
            wb = plsc.load_gather(wbot_v, [pv])
            fxp = plsc.load_gather(fx_v, [pv])
            om = 1.0 - fxp
            for cg in range(CGROUPS):
                cs = pl.ds(cg * 16, 16)
                a = r00_v[p, cs]
                bv = r01_v[p, cs]
                cc = r10_v[p, cs]
                d = r11_v[p, cs]
                out_v[pl.ds(p * C + cg * 16, 16)] = (
                    wt * (a * om + bv * fxp) + wb * (cc * om + d * fxp))
            return c

        lax.fori_loop(0, P, combine, 0)
        pltpu.sync_copy(out_v, out_hbm.at[pl.ds((base + k * P) * C, P * C)])
        return carry

    lax.fori_loop(0, NCHUNK, chunk, 0)


@functools.partial(jax.jit, static_argnames=())
def kernel(images, theta):
    table = images.reshape(NPIX, C)
    theta_flat = theta.reshape(B * 6)
    mesh = plsc.VectorSubcoreMesh(core_axis_name="c", subcore_axis_name="s")
    k = functools.partial(
        pl.kernel,
        mesh=mesh,
        out_type=jax.ShapeDtypeStruct((NPIX * C,), jnp.float32),
        compiler_params=pltpu.CompilerParams(
            needs_layout_passes=False, use_tc_tiling_on_sc=False),
        scratch_types=[
            pltpu.VMEM((B * 6,), jnp.float32),
            pltpu.VMEM((P,), jnp.int32),
            pltpu.VMEM((P,), jnp.int32),
            pltpu.VMEM((P,), jnp.int32),
            pltpu.VMEM((P,), jnp.int32),
            pltpu.VMEM((P,), jnp.float32),
            pltpu.VMEM((P,), jnp.float32),
            pltpu.VMEM((P,), jnp.float32),
            pltpu.VMEM((P, C), jnp.float32),
            pltpu.VMEM((P, C), jnp.float32),
            pltpu.VMEM((P, C), jnp.float32),
            pltpu.VMEM((P, C), jnp.float32),
            pltpu.VMEM((P * C,), jnp.float32),
            pltpu.SemaphoreType.DMA,
        ],
    )(_worker_body)
    out = k(table, theta_flat)
    return out.reshape(B, H, W, C)


# speedup vs baseline: 1.4017x; 1.4017x over previous
import functools

import jax
import jax.numpy as jnp
from jax import lax
from jax.experimental import pallas as pl
from jax.experimental.pallas import tpu as pltpu
from jax.experimental.pallas import tpu_sc as plsc

B = 8
H = 224
W = 224
C = 96
NC = 2
NS = 16
NW = NC * NS
NPIX = B * H * W
PIX_PER_W = NPIX // NW
P = 112
NCHUNK = PIX_PER_W // P
GROUPS = P // 16
CGROUPS = C // 16

_GDN = lax.GatherDimensionNumbers(
    offset_dims=(), collapsed_slice_dims=(0,), start_index_map=(0,))


def _bcast_lane(vec, lane):
    idx = jnp.full((16, 1), lane, jnp.int32)
    return lax.gather(vec, idx, _GDN, (1,),
                      mode=lax.GatherScatterMode.PROMISE_IN_BOUNDS)


def _worker_body(table_hbm, theta_hbm, out_hbm,
                 theta_v, idx_v, w_v, r_v, out_v, sems):
    wid = lax.axis_index("s") * NC + lax.axis_index("c")
    base = wid * PIX_PER_W
    b = base // (H * W)
    bb = b * (H * W)
    local0 = base - bb

    pltpu.sync_copy(theta_hbm, theta_v)
    tb = b * 6

    def bcast_theta(k):
        t = plsc.load_gather(theta_v, [jnp.full((16,), tb + k, jnp.int32)])
        u = plsc.bitcast(t, jnp.int32)
        r = (u + 0x7FFF + ((u >> 16) & 1)) & jnp.int32(-65536)
        return plsc.bitcast(r, jnp.float32)

    t0, t1, t2 = bcast_theta(0), bcast_theta(1), bcast_theta(2)
    t3, t4, t5 = bcast_theta(3), bcast_theta(4), bcast_theta(5)
    lanes_f = lax.iota(jnp.int32, 16).astype(jnp.float32)

    def stage(k, buf):
        s0 = local0 + k * P
        for g in range(GROUPS):
            s = s0 + g * 16
            i = s // W
            jb = s - i * W
            i_f = jnp.full((16,), i.astype(jnp.float32))
            j_f = jnp.full((16,), jb.astype(jnp.float32)) + lanes_f
            ys = i_f * t0 + j_f * t1 + t2
            xs = i_f * t3 + j_f * t4 + t5
            inb = ((ys >= 0.0) & (ys <= float(H - 1))
                   & (xs >= 0.0) & (xs <= float(W - 1)))
            m = jnp.where(inb, 1.0, 0.0).astype(jnp.float32)
            yc = jnp.minimum(jnp.maximum(ys, 0.0), float(H - 1))
            xc = jnp.minimum(jnp.maximum(xs, 0.0), float(W - 1))
            yb = jnp.minimum(yc.astype(jnp.int32), H - 2)
            xb = jnp.minimum(xc.astype(jnp.int32), W - 2)
            fy = yc - yb.astype(jnp.float32)
            fx = xc - xb.astype(jnp.float32)
            i00 = bb + yb * W + xb
            sl = pl.ds(g * 16, 16)
            idx_v[buf, 0, sl] = i00
            idx_v[buf, 1, sl] = i00 + 1
            idx_v[buf, 2, sl] = i00 + W
            idx_v[buf, 3, sl] = i00 + W + 1
            w_v[buf, 0, sl] = m * (1.0 - fy)
            w_v[buf, 1, sl] = m * fy
            w_v[buf, 2, sl] = fx
        copies = [pltpu.async_copy(table_hbm.at[idx_v.at[buf, c]],
                                   r_v.at[buf, c], sems.at[buf])
                  for c in range(4)]
        return copies

    def drain(buf):
        for c in range(4):
            pltpu.make_async_copy(table_hbm.at[idx_v.at[buf, c]],
                                  r_v.at[buf, c], sems.at[buf]).wait()

    def combine_and_store(k, buf):
        def group(g, carry):
            wtg = w_v[buf, 0, pl.ds(g * 16, 16)]
            wbg = w_v[buf, 1, pl.ds(g * 16, 16)]
            fxg = w_v[buf, 2, pl.ds(g * 16, 16)]
            for l in range(16):
                p = g * 16 + l
                wt = _bcast_lane(wtg, l)
                wb = _bcast_lane(wbg, l)
                fxp = _bcast_lane(fxg, l)
                om = 1.0 - fxp
                for cg in range(CGROUPS):
                    cs = pl.ds(cg * 16, 16)
                    a = r_v[buf, 0, p, cs]
                    bv = r_v[buf, 1, p, cs]
                    cc = r_v[buf, 2, p, cs]
                    d = r_v[buf, 3, p, cs]
                    out_v[pl.ds(p * C + cg * 16, 16)] = (
                        wt * (a * om + bv * fxp) + wb * (cc * om + d * fxp))
            return carry

        lax.fori_loop(0, GROUPS, group, 0)
        pltpu.sync_copy(out_v, out_hbm.at[pl.ds((base + k * P) * C, P * C)])

    stage(0, 0)

    def two_chunks(k2, carry):
        k0 = k2 * 2
        for par in (0, 1):
            k = k0 + par
            nxt = 1 - par

            @pl.when(k + 1 < NCHUNK)
            def _():
                stage(k + 1, nxt)

            drain(par)
            combine_and_store(k, par)
        return carry

    lax.fori_loop(0, NCHUNK // 2, two_chunks, 0)


@functools.partial(jax.jit, static_argnames=())
def kernel(images, theta):
    table = images.reshape(NPIX, C)
    theta_flat = theta.reshape(B * 6)
    mesh = plsc.VectorSubcoreMesh(core_axis_name="c", subcore_axis_name="s")
    k = functools.partial(
        pl.kernel,
        mesh=mesh,
        out_type=jax.ShapeDtypeStruct((NPIX * C,), jnp.float32),
        compiler_params=pltpu.CompilerParams(
            needs_layout_passes=False, use_tc_tiling_on_sc=False),
        scratch_types=[
            pltpu.VMEM((B * 6,), jnp.float32),
            pltpu.VMEM((2, 4, P), jnp.int32),
            pltpu.VMEM((2, 3, P), jnp.float32),
            pltpu.VMEM((2, 4, P, C), jnp.float32),
            pltpu.VMEM((P * C,), jnp.float32),
            pltpu.SemaphoreType.DMA((2,)),
        ],
    )(_worker_body)
    out = k(table, theta_flat)
    return out.reshape(B, H, W, C)
